# baseline (speedup 1.0000x reference)
import functools

import jax
import jax.numpy as jnp
from jax import lax
from jax.experimental import pallas as pl
from jax.experimental.pallas import tpu as pltpu
from jax.experimental.pallas import tpu_sc as plsc

B = 16384
L = 20
NBUCKETS = 100000
NFEAT = 6

NC = 2
NS = 16
NW = NC * NS

HCOLS = 128
CHUNK = 128
CPF = B // CHUNK
NCHUNKS = NFEAT * CPF
CPW = NCHUNKS // NW
GPC = CHUNK // 16

ROWBLK = 1024


def _hash_u32(x):
    h = x.astype(jnp.uint32)
    h = h * jnp.uint32(2654435761)
    h = h ^ (h >> 16)
    h = h * jnp.uint32(2246822519)
    h = h ^ (h >> 13)
    return (h % jnp.uint32(NBUCKETS)).astype(jnp.int32)


def _hash_tc_body(*refs):
    id_refs = refs[:NFEAT]
    out_ref = refs[NFEAT]
    parts = [_hash_u32(r[:, :]) for r in id_refs]
    parts.append(jnp.zeros((ROWBLK, HCOLS - NFEAT * L), jnp.int32))
    out_ref[:, :] = jnp.concatenate(parts, axis=1)


@jax.jit
def _hash_tc(*ids):
    return pl.pallas_call(
        _hash_tc_body,
        grid=(B // ROWBLK,),
        in_specs=[pl.BlockSpec((ROWBLK, L), lambda i: (i, 0))] * NFEAT,
        out_specs=pl.BlockSpec((ROWBLK, HCOLS), lambda i: (i, 0)),
        out_shape=jax.ShapeDtypeStruct((B, HCOLS), jnp.int32),
    )(*ids)


def _sc_body(*refs):
    h_hbm = refs[0]
    w_refs = refs[1:1 + NFEAT]
    part_hbm = refs[1 + NFEAT]
    table_v, h_v, out_v = refs[2 + NFEAT:]

    wid = lax.axis_index("c") * NS + lax.axis_index("s")
    c_lo = wid * CPW
    c_hi = c_lo + CPW

    def load_table(f):
        for i in range(NFEAT):
            @pl.when(f == i)
            def _load():
                pltpu.sync_copy(w_refs[i], table_v)

    def do_chunk(c, _):
        f = c // CPF
        r0 = (c % CPF) * CHUNK
        pltpu.sync_copy(h_hbm.at[pl.ds(r0, CHUNK)], h_v)
        col0 = f * L

        def group(g, _):
            rows = g * 16 + lax.iota(jnp.int32, 16)
            hs = []
            acc = jnp.zeros((16,), jnp.float32)
            for j in range(L):
                h = plsc.load_gather(h_v, [rows, col0 + j + jnp.zeros((16,), jnp.int32)])
                wj = plsc.load_gather(table_v, [h])
                if j == 0:
                    acc = wj
                else:
                    hu = h.astype(jnp.uint32)
                    md = hs[0] ^ hu
                    for k in range(1, j):
                        md = jnp.minimum(md, hs[k] ^ hu)
                    acc = acc + jnp.where(md != 0, wj, 0.0)
                hs.append(h.astype(jnp.uint32))
            out_v[pl.ds(g * 16, 16)] = acc
            return 0

        lax.fori_loop(0, GPC, group, 0)
        pltpu.sync_copy(out_v, part_hbm.at[f, pl.ds(r0, CHUNK)])
        return 0

    f0 = c_lo // CPF
    f1 = (c_hi - 1) // CPF
    split = jnp.minimum(c_hi, (f0 + 1) * CPF)

    load_table(f0)
    lax.fori_loop(c_lo, split, do_chunk, 0)

    @pl.when(f1 != f0)
    def _second_feature():
        load_table(f1)
        lax.fori_loop(split, c_hi, do_chunk, 0)


@jax.jit
def _sc_partials(h_all, *ws):
    mesh = plsc.VectorSubcoreMesh(core_axis_name="c", subcore_axis_name="s")
    return pl.kernel(
        _sc_body,
        out_type=jax.ShapeDtypeStruct((NFEAT, B), jnp.float32),
        mesh=mesh,
        scratch_types=[
            pltpu.VMEM((NBUCKETS,), jnp.float32),
            pltpu.VMEM((CHUNK, HCOLS), jnp.int32),
            pltpu.VMEM((CHUNK,), jnp.float32),
        ],
        compiler_params=pltpu.CompilerParams(needs_layout_passes=False),
    )(h_all, *ws)


def _epilogue_body(part_ref, bias_ref, out_ref):
    out_ref[:, :] = jnp.sum(part_ref[:, :], axis=0, keepdims=True) + bias_ref[0, 0]


@jax.jit
def _epilogue(part, bias):
    out = pl.pallas_call(
        _epilogue_body,
        out_shape=jax.ShapeDtypeStruct((1, B), jnp.float32),
    )(part, bias.reshape(1, 1))
    return out.reshape(B, 1)


def kernel(user_id, item_id, category_id, shop_id, hist_item_id, target_item_id,
           w_user_id, w_item_id, w_category_id, w_shop_id, w_hist_item_id,
           w_target_item_id, bias):
    ids = [user_id, item_id, category_id, shop_id, hist_item_id, target_item_id]
    ids = [x.astype(jnp.int32) for x in ids]
    ws = [w_user_id, w_item_id, w_category_id, w_shop_id, w_hist_item_id,
          w_target_item_id]
    h_all = _hash_tc(*ids)
    part = _sc_partials(h_all, *ws)
    return _epilogue(part, bias)


# speedup vs baseline: 1.0812x; 1.0812x over previous
import functools

import jax
import jax.numpy as jnp
from jax import lax
from jax.experimental import pallas as pl
from jax.experimental.pallas import tpu as pltpu
from jax.experimental.pallas import tpu_sc as plsc

B = 16384
L = 20
NBUCKETS = 100000
TROWS = 800
NFEAT = 6

NC = 2
NS = 16
NW = NC * NS

CHUNK = 128
CPF = B // CHUNK
NCHUNKS = NFEAT * CPF
CPW = NCHUNKS // NW
GPC = CHUNK // 16


def _hash16(x):
    h = x.astype(jnp.uint32)
    h = h * jnp.uint32(2654435761)
    h = h ^ (h >> 16)
    h = h * jnp.uint32(2246822519)
    h = h ^ (h >> 13)
    return h % jnp.uint32(NBUCKETS)


def _sc_body(*refs):
    ids_refs = refs[0:NFEAT]
    w_refs = refs[NFEAT:2 * NFEAT]
    part_hbm = refs[2 * NFEAT]
    table_v, ids_v, out_v = refs[2 * NFEAT + 1:]

    wid = lax.axis_index("c") * NS + lax.axis_index("s")
    c_lo = wid * CPW
    c_hi = c_lo + CPW

    def load_table(f):
        for i in range(NFEAT):
            @pl.when(f == i)
            def _load():
                pltpu.sync_copy(w_refs[i], table_v)

    def do_chunk(c, _):
        f = c // CPF
        r0 = (c % CPF) * CHUNK
        for i in range(NFEAT):
            @pl.when(f == i)
            def _load_ids():
                pltpu.sync_copy(ids_refs[i].at[pl.ds(r0, CHUNK), :], ids_v)

        def group(g, _):
            rows = g * 16 + lax.iota(jnp.int32, 16)
            hs = []
            acc = jnp.zeros((16,), jnp.float32)
            for j in range(L):
                idj = plsc.load_gather(ids_v, [rows, jnp.full((16,), j, jnp.int32)])
                h = _hash16(idj)
                hi = h.astype(jnp.int32)
                wj = plsc.load_gather(table_v, [hi >> 7, hi & 127])
                if j == 0:
                    acc = wj
                else:
                    md = hs[0] ^ h
                    for k in range(1, j):
                        md = jnp.minimum(md, hs[k] ^ h)
                    acc = acc + jnp.where(md != 0, wj, 0.0)
                hs.append(h)
            out_v[pl.ds(g * 16, 16)] = acc
            return 0

        lax.fori_loop(0, GPC, group, 0)
        pltpu.sync_copy(out_v, part_hbm.at[f, pl.ds(r0, CHUNK)])
        return 0

    f0 = c_lo // CPF
    f1 = (c_hi - 1) // CPF
    split = jnp.minimum(c_hi, (f0 + 1) * CPF)

    load_table(f0)
    lax.fori_loop(c_lo, split, do_chunk, 0)

    @pl.when(f1 != f0)
    def _second_feature():
        load_table(f1)
        lax.fori_loop(split, c_hi, do_chunk, 0)


@jax.jit
def _sc_partials(*arrays):
    mesh = plsc.VectorSubcoreMesh(core_axis_name="c", subcore_axis_name="s")
    return pl.kernel(
        _sc_body,
        out_type=jax.ShapeDtypeStruct((NFEAT, B), jnp.float32),
        mesh=mesh,
        scratch_types=[
            pltpu.VMEM((TROWS, 128), jnp.float32),
            pltpu.VMEM((CHUNK, L), jnp.int32),
            pltpu.VMEM((CHUNK,), jnp.float32),
        ],
        compiler_params=pltpu.CompilerParams(needs_layout_passes=False),
    )(*arrays)


def _epilogue_body(part_ref, bias_ref, out_ref):
    out_ref[:, :] = jnp.sum(part_ref[:, :], axis=0, keepdims=True) + bias_ref[0, 0]


@jax.jit
def _epilogue(part, bias):
    out = pl.pallas_call(
        _epilogue_body,
        out_shape=jax.ShapeDtypeStruct((1, B), jnp.float32),
    )(part, bias.reshape(1, 1))
    return out.reshape(B, 1)


def kernel(user_id, item_id, category_id, shop_id, hist_item_id, target_item_id,
           w_user_id, w_item_id, w_category_id, w_shop_id, w_hist_item_id,
           w_target_item_id, bias):
    ids = [user_id, item_id, category_id, shop_id, hist_item_id, target_item_id]
    ids = [x.astype(jnp.int32) for x in ids]
    ws = [w_user_id, w_item_id, w_category_id, w_shop_id, w_hist_item_id,
          w_target_item_id]
    ws = [jnp.pad(w, (0, TROWS * 128 - NBUCKETS)).reshape(TROWS, 128) for w in ws]
    part = _sc_partials(*ids, *ws)
    return _epilogue(part, bias)
